# Initial kernel scaffold; baseline (speedup 1.0000x reference)
import jax
import jax.numpy as jnp
from jax.experimental import pallas as pl


def kernel(num_real_unit, cmd_type, target_type, x, y, target_attack_idx, target_gather_idx, enemy_feat, resource_feat, cmd_emb, tgt_emb, v_xy, g_xy, b_xy, v_e, g_e, b_e, v_r, g_r, b_r):
    raise NotImplementedError("write your pallas kernel here")


# speedup vs baseline: 1.4424x; 1.4424x over previous
import functools

import jax
import jax.numpy as jnp
from jax import lax
from jax.experimental import pallas as pl
from jax.experimental.pallas import tpu as pltpu

B = 1024
P = 50
NE = 50
NR = 50
NUM_CMD = 7
NUM_TGT = 1000
TFD = 256
AD = 128
GATHER = 1
ATTACK = 2
BUILD_BUILDING = 3
BUILD_UNIT = 4
MOVE = 5

BB = 8
RB = BB * P
GRID = B // BB


def _body(cmd_ref, tgt_ref, gie_ref, gir_ref, xy_ref, ef_ref, rf_ref,
          cemb_ref, temb_ref, vxy_ref, gxy_ref, bxy_ref,
          ve_ref, ge_ref, be_ref, vr_ref, gr_ref, br_ref, out_ref):
    f32 = jnp.float32
    cmd = cmd_ref[...]
    tt_b = (cmd == BUILD_BUILDING) | (cmd == BUILD_UNIT)
    xy_m = ((cmd == BUILD_BUILDING) | (cmd == MOVE)).astype(f32)
    e_m = (cmd == ATTACK).astype(f32)
    r_m = (cmd == GATHER).astype(f32)

    ioc = lax.broadcasted_iota(jnp.int32, (RB, NUM_CMD), 1)
    oh_c = (cmd == ioc).astype(f32)
    out_ref[:, 0:AD] = jnp.dot(oh_c, cemb_ref[...], preferred_element_type=f32)

    tgt = tgt_ref[...]
    iot = lax.broadcasted_iota(jnp.int32, (RB, NUM_TGT), 1)
    oh_t = ((tgt == iot) & tt_b).astype(f32)
    out_ref[:, AD:2 * AD] = jnp.dot(oh_t, temb_ref[...],
                                    preferred_element_type=f32)

    vxy = vxy_ref[...]
    sxy = gxy_ref[0, 0] / jnp.sqrt(jnp.sum(vxy * vxy))
    xyf = lax.dot_general(xy_ref[...], vxy, (((1,), (1,)), ((), ())),
                          preferred_element_type=f32) * sxy + bxy_ref[...]
    out_ref[:, 2 * AD:3 * AD] = xy_m * xyf

    ve = ve_ref[...]
    se = ge_ref[0, 0] / jnp.sqrt(jnp.sum(ve * ve))
    vr = vr_ref[...]
    sr = gr_ref[0, 0] / jnp.sqrt(jnp.sum(vr * vr))
    ef2 = ef_ref[...].reshape(RB, TFD)
    rf2 = rf_ref[...].reshape(RB, TFD)
    efw = lax.dot_general(ef2, ve, (((1,), (1,)), ((), ())),
                          preferred_element_type=f32) * se
    rfw = lax.dot_general(rf2, vr, (((1,), (1,)), ((), ())),
                          preferred_element_type=f32) * sr
    ior = lax.broadcasted_iota(jnp.int32, (RB, RB), 1)
    oh_e = (gie_ref[...] == ior).astype(f32) * e_m
    oh_r = (gir_ref[...] == ior).astype(f32) * r_m
    out4 = (jnp.dot(oh_e, efw, preferred_element_type=f32)
            + jnp.dot(oh_r, rfw, preferred_element_type=f32)
            + e_m * be_ref[...] + r_m * br_ref[...])
    out_ref[:, 3 * AD:4 * AD] = out4


@jax.jit
def _run(cmd, tgt, gie, gir, xy, enemy_feat, resource_feat,
         cmd_emb, tgt_emb, v_xy, g_xy, b_xy, v_e, g_e, b_e, v_r, g_r, b_r):
    row = lambda: pl.BlockSpec((RB, 1), lambda i: (i, 0))
    full = lambda a, b: pl.BlockSpec((a, b), lambda i: (0, 0))
    return pl.pallas_call(
        _body,
        grid=(GRID,),
        in_specs=[
            row(),
            row(),
            row(),
            row(),
            pl.BlockSpec((RB, 2), lambda i: (i, 0)),
            pl.BlockSpec((BB, NE, TFD), lambda i: (i, 0, 0)),
            pl.BlockSpec((BB, NR, TFD), lambda i: (i, 0, 0)),
            full(NUM_CMD, AD),
            full(NUM_TGT, AD),
            full(AD, 2),
            full(1, 1),
            full(1, AD),
            full(AD, TFD),
            full(1, 1),
            full(1, AD),
            full(AD, TFD),
            full(1, 1),
            full(1, AD),
        ],
        out_specs=pl.BlockSpec((RB, 4 * AD), lambda i: (i, 0)),
        out_shape=jax.ShapeDtypeStruct((B * P, 4 * AD), jnp.float32),
    )(cmd, tgt, gie, gir, xy, enemy_feat, resource_feat,
      cmd_emb, tgt_emb, v_xy, g_xy, b_xy, v_e, g_e, b_e, v_r, g_r, b_r)


def kernel(num_real_unit, cmd_type, target_type, x, y, target_attack_idx,
           target_gather_idx, enemy_feat, resource_feat,
           cmd_emb, tgt_emb, v_xy, g_xy, b_xy, v_e, g_e, b_e, v_r, g_r, b_r):
    del num_real_unit
    i32 = jnp.int32
    cmd = cmd_type.astype(i32).reshape(B * P, 1)
    tgt = target_type.astype(i32).reshape(B * P, 1)
    boff = (jnp.arange(B, dtype=i32) % BB)[:, None] * NE
    gie = (boff + target_attack_idx.astype(i32)).reshape(B * P, 1)
    gir = (boff + target_gather_idx.astype(i32)).reshape(B * P, 1)
    xy = jnp.stack([x, y], axis=2).reshape(B * P, 2)
    out = _run(cmd, tgt, gie, gir, xy, enemy_feat, resource_feat,
               cmd_emb, tgt_emb, v_xy,
               g_xy.reshape(1, 1), b_xy.reshape(1, AD),
               v_e, g_e.reshape(1, 1), b_e.reshape(1, AD),
               v_r, g_r.reshape(1, 1), b_r.reshape(1, AD))
    return out.reshape(B, P, 4 * AD)
